# Initial kernel scaffold; baseline (speedup 1.0000x reference)
import jax
import jax.numpy as jnp
from jax.experimental import pallas as pl


def kernel(logit, temperature):
    raise NotImplementedError("write your pallas kernel here")


# speedup vs baseline: 1.2078x; 1.2078x over previous
import math
import jax
import jax.numpy as jnp
from jax.experimental import pallas as pl

_M = 4
_K = 512
_INV_SCALE = 1.0 / math.sqrt(_K)


def _body(x_ref, q_ref, l_ref):
    x = x_ref[0]
    xs = x * _INV_SCALE
    l_ref[0] = xs
    m = jnp.max(xs, axis=0, keepdims=True)
    rio = jax.lax.broadcasted_iota(jnp.int32, xs.shape, 0)
    cand = jnp.where(xs == m, rio, _K)
    idx = jnp.min(cand, axis=0, keepdims=True)
    q_ref[0] = jnp.where(rio == idx, 1.0, 0.0).astype(xs.dtype)


def kernel(logit, temperature):
    n, c, h, w = logit.shape
    g = c // _M
    hw = h * w
    nb = n * _M
    x = logit.reshape(nb, g, hw)

    blk = (1, g, hw)
    q, l = pl.pallas_call(
        _body,
        grid=(nb,),
        in_specs=[pl.BlockSpec(blk, lambda i: (i, 0, 0))],
        out_specs=[
            pl.BlockSpec(blk, lambda i: (i, 0, 0)),
            pl.BlockSpec(blk, lambda i: (i, 0, 0)),
        ],
        out_shape=[
            jax.ShapeDtypeStruct((nb, g, hw), logit.dtype),
            jax.ShapeDtypeStruct((nb, g, hw), logit.dtype),
        ],
    )(x)
    return q.reshape(n, c, h, w), l.reshape(n, _M, g, h, w)
